# baseline (speedup 1.0000x reference)
import functools

import numpy as np
import jax
import jax.numpy as jnp
from jax import lax
from jax.experimental import pallas as pl
from jax.experimental.pallas import tpu as pltpu

_N = 1024
_INF = float("inf")
_IMIN = -(2 ** 31)
_IMAX = 2 ** 31 - 1


_DIAGINF = np.zeros((_N, _N), np.float32)
np.fill_diagonal(_DIAGINF, np.inf)


def _two_next_smallest(d2b):
    m2 = jnp.min(d2b, axis=0)
    idx2 = jnp.argmin(d2b, axis=0)
    rows = lax.broadcasted_iota(jnp.int32, (_N, _N), 0)
    d2c = jnp.where(rows == idx2[None, :], _INF, d2b)
    m3 = jnp.min(d2c, axis=0)
    return m2, m3


def _d2mat(x, y):
    dx = x.reshape(_N, 1) - x.reshape(1, _N)
    dy = y.reshape(_N, 1) - y.reshape(1, _N)
    return dx * dx + dy * dy


def _chunked(m2, m3):
    return jnp.concatenate([m2.reshape(8, 128), m3.reshape(8, 128)], axis=0)


def _dist_body(fx_ref, fy_ref, fr_ref, rx_ref, ry_ref, dinf_ref,
               knnf_ref, knnr_ref, feas_ref, sumr_ref):
    dinf = dinf_ref[...]

    fx = fx_ref[0, 0, :]
    fy = fy_ref[0, 0, :]
    d2f = _d2mat(fx, fy)

    m2, m3 = _two_next_smallest(d2f + dinf)
    knnf_ref[:, 0, 0, :] = jnp.sqrt(_chunked(m2, m3))

    dist = jnp.sqrt(d2f)
    r = jnp.abs(fr_ref[0, 0, :])
    radiim = (r - 0.0001).reshape(_N, 1) + r.reshape(1, _N)
    ov = jnp.maximum(radiim - dist, 0.0)
    diag_ov = jnp.sum(jnp.maximum((r - 0.0001) + r, 0.0))
    total = jnp.sum(ov) - diag_ov
    feas_ref[0, 0, :] = jnp.full((128,), 0.5 * total)
    sumr_ref[0, 0, :] = jnp.full((128,), jnp.sum(r))

    d2r = _d2mat(rx_ref[0, 0, :], ry_ref[0, 0, :])
    m2, m3 = _two_next_smallest(d2r + dinf)
    knnr_ref[:, 0, 0, :] = jnp.sqrt(_chunked(m2, m3))


def _dist_call(fx, fy, fr, rx, ry, interpret=False):
    B = fx.shape[0]
    row = pl.BlockSpec((1, 1, _N), lambda b: (b, 0, 0))
    full = pl.BlockSpec((_N, _N), lambda b: (0, 0))
    knn = pl.BlockSpec((16, 1, 1, 128), lambda b: (0, b, 0, 0))
    lane = pl.BlockSpec((1, 1, 128), lambda b: (b, 0, 0))
    outs = [jax.ShapeDtypeStruct((16, B, 1, 128), jnp.float32)] * 2 + \
           [jax.ShapeDtypeStruct((B, 1, 128), jnp.float32)] * 2
    ins = [a.reshape(B, 1, _N) for a in (fx, fy, fr, rx, ry)] + \
          [jnp.asarray(_DIAGINF)]
    knnf, knnr, feas, sumr = pl.pallas_call(
        _dist_body,
        grid=(B,),
        in_specs=[row] * 5 + [full],
        out_specs=[knn] * 2 + [lane] * 2,
        out_shape=outs,
        compiler_params=pltpu.CompilerParams(
            dimension_semantics=("arbitrary",)),
        interpret=interpret,
    )(*ins)
    return (knnf.reshape(16, B, 128), knnr.reshape(16, B, 128),
            feas.reshape(B, 128), sumr.reshape(B, 128))


def _qpos(q, n):
    pos = np.float32(q) * np.float32(n - 1)
    lo = int(np.floor(pos))
    return lo, float(pos - np.float32(lo))


def _to_ukey(f):
    b = lax.bitcast_convert_type(f, jnp.int32)
    key = b ^ ((b >> 31) & jnp.int32(0x7FFFFFFF))
    return key ^ jnp.int32(_IMIN)


def _key_to_f32(key):
    b = key ^ ((key >> 31) & jnp.int32(0x7FFFFFFF))
    return lax.bitcast_convert_type(b, jnp.float32)


def _bisect(data_u, ranks, count):
    def step(pi, carry):
        p = 31 - pi
        res, rem = carry
        sp = jnp.right_shift(data_u, p)
        bit = jnp.left_shift(jnp.int32(1), p)
        nres, nrem = [], []
        for r, m in zip(res, rem):
            cnt = count(sp == jnp.right_shift(r, p))
            go1 = m >= cnt
            nres.append(jnp.where(go1, jnp.bitwise_or(r, bit), r))
            nrem.append(jnp.where(go1, m - cnt, m))
        return tuple(nres), tuple(nrem)

    res0 = tuple(jnp.zeros_like(r) for r in ranks)
    res, _ = lax.fori_loop(0, 32, step, (res0, tuple(ranks)))
    return list(res)


def _pair_from_lo(skeys, ukey_lo, lo_rank, count, reduce_min):
    klo = ukey_lo ^ jnp.int32(_IMIN)
    cnt = count(skeys <= klo)
    succ = reduce_min(jnp.where(skeys > klo, skeys, jnp.int32(_IMAX)))
    khi = jnp.where(cnt >= lo_rank + 2, klo, succ)
    return _key_to_f32(klo), _key_to_f32(khi)


def _interp(vlo, vhi, frac):
    return vlo + (vhi - vlo) * jnp.float32(frac)


_Q7 = [0.05, 0.1, 0.25, 0.5, 0.75, 0.9, 0.95]
_Q5 = [0.05, 0.25, 0.5, 0.75, 0.95]


def _channel_quantiles(data_f32, qs):
    n = _N * 16
    pos = [_qpos(q, n) for q in qs]
    data_u = _to_ukey(data_f32)
    skeys = data_u ^ jnp.int32(_IMIN)
    count = lambda pred: jnp.sum(pred.astype(jnp.int32))
    ukeys = _bisect(data_u, [jnp.int32(lo) for lo, _ in pos], count)
    out = []
    for (lo, frac), uk in zip(pos, ukeys):
        vlo, vhi = _pair_from_lo(skeys, uk, lo, count, jnp.min)
        out.append(_interp(vlo, vhi, frac))
    return out


def _knn_quantiles(knn_u):
    lo50, frac50 = _qpos(0.5, 3 * _N)
    lo95, frac95 = _qpos(0.95, 3 * _N)
    d50, d95 = lo50 - _N, lo95 - _N

    def count(pred):
        s = jnp.sum(pred.astype(jnp.int32), axis=2, keepdims=True)
        return jnp.sum(s, axis=0, keepdims=True)

    def reduce_min(x):
        s = jnp.min(x, axis=2, keepdims=True)
        return jnp.min(s, axis=0, keepdims=True)

    skeys = knn_u ^ jnp.int32(_IMIN)
    r0 = jnp.zeros((1, 16, 1), jnp.int32)
    ukeys = _bisect(knn_u, [r0 + d50, r0 + d95], count)
    v50 = _interp(*_pair_from_lo(skeys, ukeys[0], d50, count, reduce_min),
                  frac50)
    v95 = _interp(*_pair_from_lo(skeys, ukeys[1], d95, count, reduce_min),
                  frac95)
    return v50, v95


def _loss_body(fr_ref, rr_ref, fx_ref, rx_ref, fy_ref, ry_ref,
               knnf_ref, knnr_ref, feas_ref, sumr_ref, fo_ref, out_ref):
    qfr = _channel_quantiles(fr_ref[...], _Q7)
    qrr = _channel_quantiles(rr_ref[...], _Q7)
    radius_loss = sum((a - b) ** 2 for a, b in zip(qfr, qrr)) / 7.0

    qfx = _channel_quantiles(fx_ref[...], _Q5)
    qrx = _channel_quantiles(rx_ref[...], _Q5)
    qfy = _channel_quantiles(fy_ref[...], _Q5)
    qry = _channel_quantiles(ry_ref[...], _Q5)
    grid_loss = (sum((a - b) ** 2 for a, b in zip(qfx, qrx)) / 5.0
                 + sum((a - b) ** 2 for a, b in zip(qfy, qry)) / 5.0) / 2.0

    f50, f95 = _knn_quantiles(_to_ukey(knnf_ref[...]))
    r50, r95 = _knn_quantiles(_to_ukey(knnr_ref[...]))
    d50 = f50 - r50
    d95 = f95 - r95
    distance_loss = (jnp.sum(d50 * d50) + jnp.sum(d95 * d95)) / 48.0

    feas_loss = jnp.sum(feas_ref[:, 0:1]) / (
        jnp.sum(sumr_ref[:, 0:1]) * jnp.float32(_N))

    p = fo_ref[0, :]
    logp = jnp.maximum(jnp.log(p), -100.0)
    log1mp = jnp.maximum(jnp.log(1.0 - p), -100.0)
    gan_loss = -jnp.mean(0.9 * logp + 0.1 * log1mp)

    total = radius_loss + feas_loss + gan_loss + grid_loss + distance_loss
    out_ref[:, :] = jnp.full((8, 128), total)


def _loss_call(chs, knnf, knnr, feas, sumr, fo, interpret=False):
    return pl.pallas_call(
        _loss_body,
        out_shape=jax.ShapeDtypeStruct((8, 128), jnp.float32),
        interpret=interpret,
    )(*chs, knnf, knnr, feas, sumr, fo)


def kernel(real_images, fake_images, fake_outputs, interpret=False):
    B = real_images.shape[0]
    fx = fake_images[:, :, 0]
    fy = fake_images[:, :, 1]
    fr = fake_images[:, :, 2]
    rx = real_images[:, :, 0]
    ry = real_images[:, :, 1]
    rr = real_images[:, :, 2]

    knnf, knnr, feas, sumr = _dist_call(
        fx, fy, fr, rx, ry, interpret=interpret)

    out = _loss_call((fr, rr, fx, rx, fy, ry), knnf, knnr, feas, sumr,
                     fake_outputs.reshape(1, B), interpret=interpret)
    return out[0, 0]


# speedup vs baseline: 1.0628x; 1.0628x over previous
import functools

import numpy as np
import jax
import jax.numpy as jnp
from jax import lax
from jax.experimental import pallas as pl
from jax.experimental.pallas import tpu as pltpu

_N = 1024
_INF = float("inf")
_IMIN = -(2 ** 31)
_IMAX = 2 ** 31 - 1


def _two_next_smallest(d2, rows, diag):
    d2b = jnp.where(diag, _INF, d2)
    m2 = jnp.min(d2b, axis=0)
    idx2 = jnp.argmin(d2b, axis=0)
    d2c = jnp.where(rows == idx2[None, :], _INF, d2b)
    m3 = jnp.min(d2c, axis=0)
    return m2, m3


def _d2mat(x, y):
    dx = x.reshape(_N, 1) - x.reshape(1, _N)
    dy = y.reshape(_N, 1) - y.reshape(1, _N)
    return dx * dx + dy * dy


def _chunked(m2, m3):
    return jnp.concatenate([m2.reshape(8, 128), m3.reshape(8, 128)], axis=0)


def _dist_body(fx_ref, fy_ref, fr_ref, rx_ref, ry_ref,
               knnf_ref, knnr_ref, feas_ref, sumr_ref):
    rows = lax.broadcasted_iota(jnp.int32, (_N, _N), 0)
    cols = lax.broadcasted_iota(jnp.int32, (_N, _N), 1)
    diag = rows == cols

    fx = fx_ref[0, 0, :]
    fy = fy_ref[0, 0, :]
    d2f = _d2mat(fx, fy)

    m2, m3 = _two_next_smallest(d2f, rows, diag)
    knnf_ref[:, 0, 0, :] = jnp.sqrt(_chunked(m2, m3))

    dist = jnp.sqrt(d2f)
    r = jnp.abs(fr_ref[0, 0, :])
    radiim = (r - 0.0001).reshape(_N, 1) + r.reshape(1, _N)
    ov = jnp.maximum(radiim - dist, 0.0)
    diag_ov = jnp.sum(jnp.maximum((r - 0.0001) + r, 0.0))
    total = jnp.sum(ov) - diag_ov
    feas_ref[0, 0, :] = jnp.full((128,), 0.5 * total)
    sumr_ref[0, 0, :] = jnp.full((128,), jnp.sum(r))

    d2r = _d2mat(rx_ref[0, 0, :], ry_ref[0, 0, :])
    m2, m3 = _two_next_smallest(d2r, rows, diag)
    knnr_ref[:, 0, 0, :] = jnp.sqrt(_chunked(m2, m3))


def _dist_call(fx, fy, fr, rx, ry, interpret=False):
    B = fx.shape[0]
    row = pl.BlockSpec((1, 1, _N), lambda b: (b, 0, 0))
    knn = pl.BlockSpec((16, 1, 1, 128), lambda b: (0, b, 0, 0))
    lane = pl.BlockSpec((1, 1, 128), lambda b: (b, 0, 0))
    outs = [jax.ShapeDtypeStruct((16, B, 1, 128), jnp.float32)] * 2 + \
           [jax.ShapeDtypeStruct((B, 1, 128), jnp.float32)] * 2
    ins = [a.reshape(B, 1, _N) for a in (fx, fy, fr, rx, ry)]
    knnf, knnr, feas, sumr = pl.pallas_call(
        _dist_body,
        grid=(B,),
        in_specs=[row] * 5,
        out_specs=[knn] * 2 + [lane] * 2,
        out_shape=outs,
        compiler_params=pltpu.CompilerParams(
            dimension_semantics=("arbitrary",)),
        interpret=interpret,
    )(*ins)
    return (knnf.reshape(16, B, 128), knnr.reshape(16, B, 128),
            feas.reshape(B, 128), sumr.reshape(B, 128))


def _qpos(q, n):
    pos = np.float32(q) * np.float32(n - 1)
    lo = int(np.floor(pos))
    return lo, float(pos - np.float32(lo))


def _to_ukey(f):
    b = lax.bitcast_convert_type(f, jnp.int32)
    key = b ^ ((b >> 31) & jnp.int32(0x7FFFFFFF))
    return key ^ jnp.int32(_IMIN)


def _key_to_f32(key):
    b = key ^ ((key >> 31) & jnp.int32(0x7FFFFFFF))
    return lax.bitcast_convert_type(b, jnp.float32)


def _bisect(data_u, ranks, count):
    def step(pi, carry):
        p = 31 - pi
        res, rem = carry
        sp = jnp.right_shift(data_u, p)
        bit = jnp.left_shift(jnp.int32(1), p)
        nres, nrem = [], []
        for r, m in zip(res, rem):
            cnt = count(sp == jnp.right_shift(r, p))
            go1 = m >= cnt
            nres.append(jnp.where(go1, jnp.bitwise_or(r, bit), r))
            nrem.append(jnp.where(go1, m - cnt, m))
        return tuple(nres), tuple(nrem)

    res0 = tuple(jnp.zeros_like(r) for r in ranks)
    res, _ = lax.fori_loop(0, 32, step, (res0, tuple(ranks)))
    return list(res)


def _pair_from_lo(skeys, ukey_lo, lo_rank, count, reduce_min):
    klo = ukey_lo ^ jnp.int32(_IMIN)
    cnt = count(skeys <= klo)
    succ = reduce_min(jnp.where(skeys > klo, skeys, jnp.int32(_IMAX)))
    khi = jnp.where(cnt >= lo_rank + 2, klo, succ)
    return _key_to_f32(klo), _key_to_f32(khi)


def _interp(vlo, vhi, frac):
    return vlo + (vhi - vlo) * jnp.float32(frac)


_Q7 = [0.05, 0.1, 0.25, 0.5, 0.75, 0.9, 0.95]
_Q5 = [0.05, 0.25, 0.5, 0.75, 0.95]


def _channel_quantiles(data_f32, qs):
    n = _N * 16
    pos = [_qpos(q, n) for q in qs]
    data_u = _to_ukey(data_f32)
    skeys = data_u ^ jnp.int32(_IMIN)
    count = lambda pred: jnp.sum(pred.astype(jnp.int32))
    ukeys = _bisect(data_u, [jnp.int32(lo) for lo, _ in pos], count)
    out = []
    for (lo, frac), uk in zip(pos, ukeys):
        vlo, vhi = _pair_from_lo(skeys, uk, lo, count, jnp.min)
        out.append(_interp(vlo, vhi, frac))
    return out


def _knn_quantiles(knn_u):
    lo50, frac50 = _qpos(0.5, 3 * _N)
    lo95, frac95 = _qpos(0.95, 3 * _N)
    d50, d95 = lo50 - _N, lo95 - _N

    def count(pred):
        s = jnp.sum(pred.astype(jnp.int32), axis=2, keepdims=True)
        return jnp.sum(s, axis=0, keepdims=True)

    def reduce_min(x):
        s = jnp.min(x, axis=2, keepdims=True)
        return jnp.min(s, axis=0, keepdims=True)

    skeys = knn_u ^ jnp.int32(_IMIN)
    r0 = jnp.zeros((1, 16, 1), jnp.int32)
    ukeys = _bisect(knn_u, [r0 + d50, r0 + d95], count)
    v50 = _interp(*_pair_from_lo(skeys, ukeys[0], d50, count, reduce_min),
                  frac50)
    v95 = _interp(*_pair_from_lo(skeys, ukeys[1], d95, count, reduce_min),
                  frac95)
    return v50, v95


def _loss_body(fr_ref, rr_ref, fx_ref, rx_ref, fy_ref, ry_ref,
               knnf_ref, knnr_ref, feas_ref, sumr_ref, fo_ref, out_ref):
    qfr = _channel_quantiles(fr_ref[...], _Q7)
    qrr = _channel_quantiles(rr_ref[...], _Q7)
    radius_loss = sum((a - b) ** 2 for a, b in zip(qfr, qrr)) / 7.0

    qfx = _channel_quantiles(fx_ref[...], _Q5)
    qrx = _channel_quantiles(rx_ref[...], _Q5)
    qfy = _channel_quantiles(fy_ref[...], _Q5)
    qry = _channel_quantiles(ry_ref[...], _Q5)
    grid_loss = (sum((a - b) ** 2 for a, b in zip(qfx, qrx)) / 5.0
                 + sum((a - b) ** 2 for a, b in zip(qfy, qry)) / 5.0) / 2.0

    f50, f95 = _knn_quantiles(_to_ukey(knnf_ref[...]))
    r50, r95 = _knn_quantiles(_to_ukey(knnr_ref[...]))
    d50 = f50 - r50
    d95 = f95 - r95
    distance_loss = (jnp.sum(d50 * d50) + jnp.sum(d95 * d95)) / 48.0

    feas_loss = jnp.sum(feas_ref[:, 0:1]) / (
        jnp.sum(sumr_ref[:, 0:1]) * jnp.float32(_N))

    p = fo_ref[0, :]
    logp = jnp.maximum(jnp.log(p), -100.0)
    log1mp = jnp.maximum(jnp.log(1.0 - p), -100.0)
    gan_loss = -jnp.mean(0.9 * logp + 0.1 * log1mp)

    total = radius_loss + feas_loss + gan_loss + grid_loss + distance_loss
    out_ref[:, :] = jnp.full((8, 128), total)


def _loss_call(chs, knnf, knnr, feas, sumr, fo, interpret=False):
    return pl.pallas_call(
        _loss_body,
        out_shape=jax.ShapeDtypeStruct((8, 128), jnp.float32),
        interpret=interpret,
    )(*chs, knnf, knnr, feas, sumr, fo)


def kernel(real_images, fake_images, fake_outputs, interpret=False):
    B = real_images.shape[0]
    fx = fake_images[:, :, 0]
    fy = fake_images[:, :, 1]
    fr = fake_images[:, :, 2]
    rx = real_images[:, :, 0]
    ry = real_images[:, :, 1]
    rr = real_images[:, :, 2]

    knnf, knnr, feas, sumr = _dist_call(
        fx, fy, fr, rx, ry, interpret=interpret)

    out = _loss_call((fr, rr, fx, rx, fy, ry), knnf, knnr, feas, sumr,
                     fake_outputs.reshape(1, B), interpret=interpret)
    return out[0, 0]


# speedup vs baseline: 1.0629x; 1.0001x over previous
import functools

import numpy as np
import jax
import jax.numpy as jnp
from jax import lax
from jax.experimental import pallas as pl
from jax.experimental.pallas import tpu as pltpu

_N = 1024
_INF = float("inf")
_IMIN = -(2 ** 31)
_IMAX = 2 ** 31 - 1


def _two_next_smallest(d2, rows, diag):
    d2b = jnp.where(diag, _INF, d2)
    m2 = jnp.min(d2b, axis=0)
    idx2 = jnp.argmin(d2b, axis=0)
    d2c = jnp.where(rows == idx2[None, :], _INF, d2b)
    m3 = jnp.min(d2c, axis=0)
    return m2, m3


def _d2mat(x, y):
    dx = x.reshape(_N, 1) - x.reshape(1, _N)
    dy = y.reshape(_N, 1) - y.reshape(1, _N)
    return dx * dx + dy * dy


def _chunked(m2, m3):
    return jnp.concatenate([m2.reshape(8, 128), m3.reshape(8, 128)], axis=0)


def _dist_body(fx_ref, fy_ref, fr_ref, rx_ref, ry_ref,
               knnf_ref, knnr_ref, feas_ref, sumr_ref):
    rows = lax.broadcasted_iota(jnp.int32, (_N, _N), 0)
    cols = lax.broadcasted_iota(jnp.int32, (_N, _N), 1)
    diag = rows == cols

    fx = fx_ref[0, 0, :]
    fy = fy_ref[0, 0, :]
    d2f = _d2mat(fx, fy)

    m2, m3 = _two_next_smallest(d2f, rows, diag)
    knnf_ref[:, 0, 0, :] = jnp.sqrt(_chunked(m2, m3))

    dist = jnp.sqrt(d2f)
    r = jnp.abs(fr_ref[0, 0, :])
    radiim = (r - 0.0001).reshape(_N, 1) + r.reshape(1, _N)
    ov = jnp.maximum(radiim - dist, 0.0)
    diag_ov = jnp.sum(jnp.maximum((r - 0.0001) + r, 0.0))
    total = jnp.sum(ov) - diag_ov
    feas_ref[0, 0, :] = jnp.full((128,), 0.5 * total)
    sumr_ref[0, 0, :] = jnp.full((128,), jnp.sum(r))

    d2r = _d2mat(rx_ref[0, 0, :], ry_ref[0, 0, :])
    m2, m3 = _two_next_smallest(d2r, rows, diag)
    knnr_ref[:, 0, 0, :] = jnp.sqrt(_chunked(m2, m3))


def _dist_call(fx, fy, fr, rx, ry, interpret=False):
    B = fx.shape[0]
    row = pl.BlockSpec((1, 1, _N), lambda b: (b, 0, 0))
    knn = pl.BlockSpec((16, 1, 1, 128), lambda b: (0, b, 0, 0))
    lane = pl.BlockSpec((1, 1, 128), lambda b: (b, 0, 0))
    outs = [jax.ShapeDtypeStruct((16, B, 1, 128), jnp.float32)] * 2 + \
           [jax.ShapeDtypeStruct((B, 1, 128), jnp.float32)] * 2
    ins = [a.reshape(B, 1, _N) for a in (fx, fy, fr, rx, ry)]
    knnf, knnr, feas, sumr = pl.pallas_call(
        _dist_body,
        grid=(B,),
        in_specs=[row] * 5,
        out_specs=[knn] * 2 + [lane] * 2,
        out_shape=outs,
        compiler_params=pltpu.CompilerParams(
            dimension_semantics=("arbitrary",)),
        interpret=interpret,
    )(*ins)
    return (knnf.reshape(16, B, 128), knnr.reshape(16, B, 128),
            feas.reshape(B, 128), sumr.reshape(B, 128))


def _qpos(q, n):
    pos = np.float32(q) * np.float32(n - 1)
    lo = int(np.floor(pos))
    return lo, float(pos - np.float32(lo))


def _to_ukey(f):
    b = lax.bitcast_convert_type(f, jnp.int32)
    key = b ^ ((b >> 31) & jnp.int32(0x7FFFFFFF))
    return key ^ jnp.int32(_IMIN)


def _key_to_f32(key):
    b = key ^ ((key >> 31) & jnp.int32(0x7FFFFFFF))
    return lax.bitcast_convert_type(b, jnp.float32)


def _bisect(data_u, ranks, count):
    def step(pi, carry):
        p = 31 - pi
        res, rem = carry
        sp = jnp.right_shift(data_u, p)
        bit = jnp.left_shift(jnp.int32(1), p)
        nres, nrem = [], []
        for r, m in zip(res, rem):
            cnt = count(sp == jnp.right_shift(r, p))
            go1 = m >= cnt
            nres.append(jnp.where(go1, jnp.bitwise_or(r, bit), r))
            nrem.append(jnp.where(go1, m - cnt, m))
        return tuple(nres), tuple(nrem)

    res0 = tuple(jnp.zeros_like(r) for r in ranks)
    res, _ = lax.fori_loop(0, 32, step, (res0, tuple(ranks)))
    return list(res)


def _pair_from_lo(skeys, ukey_lo, lo_rank, count, reduce_min):
    klo = ukey_lo ^ jnp.int32(_IMIN)
    cnt = count(skeys <= klo)
    succ = reduce_min(jnp.where(skeys > klo, skeys, jnp.int32(_IMAX)))
    khi = jnp.where(cnt >= lo_rank + 2, klo, succ)
    return _key_to_f32(klo), _key_to_f32(khi)


def _interp(vlo, vhi, frac):
    return vlo + (vhi - vlo) * jnp.float32(frac)


_Q7 = [0.05, 0.1, 0.25, 0.5, 0.75, 0.9, 0.95]
_Q5 = [0.05, 0.25, 0.5, 0.75, 0.95]


def _channel_quantiles(data_f32, qs):
    n = _N * 16
    pos = [_qpos(q, n) for q in qs]
    data_u = _to_ukey(data_f32)
    skeys = data_u ^ jnp.int32(_IMIN)
    count = lambda pred: jnp.sum(pred.astype(jnp.int32))
    ukeys = _bisect(data_u, [jnp.int32(lo) for lo, _ in pos], count)
    out = []
    for (lo, frac), uk in zip(pos, ukeys):
        vlo, vhi = _pair_from_lo(skeys, uk, lo, count, jnp.min)
        out.append(_interp(vlo, vhi, frac))
    return out


def _knn_quantiles(knn_u):
    lo50, frac50 = _qpos(0.5, 3 * _N)
    lo95, frac95 = _qpos(0.95, 3 * _N)
    d50, d95 = lo50 - _N, lo95 - _N

    def count(pred):
        s = jnp.sum(pred.astype(jnp.int32), axis=2, keepdims=True)
        return jnp.sum(s, axis=0, keepdims=True)

    def reduce_min(x):
        s = jnp.min(x, axis=2, keepdims=True)
        return jnp.min(s, axis=0, keepdims=True)

    skeys = knn_u ^ jnp.int32(_IMIN)
    r0 = jnp.zeros((1, 16, 1), jnp.int32)
    ukeys = _bisect(knn_u, [r0 + d50, r0 + d95], count)
    v50 = _interp(*_pair_from_lo(skeys, ukeys[0], d50, count, reduce_min),
                  frac50)
    v95 = _interp(*_pair_from_lo(skeys, ukeys[1], d95, count, reduce_min),
                  frac95)
    return v50, v95


def _loss_body(fr_ref, rr_ref, fx_ref, rx_ref, fy_ref, ry_ref,
               knnf_ref, knnr_ref, feas_ref, sumr_ref, fo_ref, out_ref):
    qfr = _channel_quantiles(fr_ref[...], _Q7)
    qrr = _channel_quantiles(rr_ref[...], _Q7)
    radius_loss = sum((a - b) ** 2 for a, b in zip(qfr, qrr)) / 7.0

    qfx = _channel_quantiles(fx_ref[...], _Q5)
    qrx = _channel_quantiles(rx_ref[...], _Q5)
    qfy = _channel_quantiles(fy_ref[...], _Q5)
    qry = _channel_quantiles(ry_ref[...], _Q5)
    grid_loss = (sum((a - b) ** 2 for a, b in zip(qfx, qrx)) / 5.0
                 + sum((a - b) ** 2 for a, b in zip(qfy, qry)) / 5.0) / 2.0

    f50, f95 = _knn_quantiles(_to_ukey(knnf_ref[...]))
    r50, r95 = _knn_quantiles(_to_ukey(knnr_ref[...]))
    d50 = f50 - r50
    d95 = f95 - r95
    distance_loss = (jnp.sum(d50 * d50) + jnp.sum(d95 * d95)) / 48.0

    feas_loss = jnp.sum(feas_ref[:, 0:1]) / (
        jnp.sum(sumr_ref[:, 0:1]) * jnp.float32(_N))

    p = fo_ref[0, :]
    logp = jnp.maximum(jnp.log(p), -100.0)
    log1mp = jnp.maximum(jnp.log(1.0 - p), -100.0)
    gan_loss = -jnp.mean(0.9 * logp + 0.1 * log1mp)

    total = radius_loss + feas_loss + gan_loss + grid_loss + distance_loss
    out_ref[:, :] = jnp.full((8, 128), total)


def _loss_call(chs, knnf, knnr, feas, sumr, fo, interpret=False):
    return pl.pallas_call(
        _loss_body,
        out_shape=jax.ShapeDtypeStruct((8, 128), jnp.float32),
        interpret=interpret,
    )(*chs, knnf, knnr, feas, sumr, fo)


def kernel(real_images, fake_images, fake_outputs, interpret=False):
    B = real_images.shape[0]
    ft = fake_images.transpose(2, 0, 1)
    rt = real_images.transpose(2, 0, 1)
    fx, fy, fr = ft[0], ft[1], ft[2]
    rx, ry, rr = rt[0], rt[1], rt[2]

    knnf, knnr, feas, sumr = _dist_call(
        fx, fy, fr, rx, ry, interpret=interpret)

    out = _loss_call((fr, rr, fx, rx, fy, ry), knnf, knnr, feas, sumr,
                     fake_outputs.reshape(1, B), interpret=interpret)
    return out[0, 0]
